# baseline (speedup 1.0000x reference)
import jax
import jax.numpy as jnp
from jax.experimental import pallas as pl
from jax.experimental.pallas import tpu as pltpu


D_MODEL = 4096
NUM_EXPERTS = 64
TILE_TOK = 1024


def _router_kernel(x_ref, w_hbm_ref, b_ref, out_ref, w_vmem, w_sem):
    @pl.when(pl.program_id(0) == 0)
    def _load_w():
        copy = pltpu.make_async_copy(w_hbm_ref, w_vmem, w_sem)
        copy.start()
        copy.wait()

    x = x_ref[...]
    w = w_vmem[...]
    logits = jax.lax.dot_general(
        x, w,
        dimension_numbers=(((1,), (1,)), ((), ())),
        preferred_element_type=jnp.float32,
    )
    logits = logits + b_ref[...]
    m = jnp.max(logits, axis=-1, keepdims=True)
    e = jnp.exp(logits - m)
    out_ref[...] = e / jnp.sum(e, axis=-1, keepdims=True)


def kernel(x, W, b):
    n_tok = x.shape[0]
    grid = (n_tok // TILE_TOK,)
    return pl.pallas_call(
        _router_kernel,
        grid=grid,
        in_specs=[
            pl.BlockSpec((TILE_TOK, D_MODEL), lambda i: (i, 0)),
            pl.BlockSpec(memory_space=pltpu.MemorySpace.HBM),
            pl.BlockSpec((NUM_EXPERTS,), lambda i: (0,)),
        ],
        out_specs=pl.BlockSpec((TILE_TOK, NUM_EXPERTS), lambda i: (i, 0)),
        out_shape=jax.ShapeDtypeStruct((n_tok, NUM_EXPERTS), jnp.float32),
        scratch_shapes=[
            pltpu.VMEM((NUM_EXPERTS, D_MODEL), jnp.float32),
            pltpu.SemaphoreType.DMA,
        ],
        compiler_params=pltpu.CompilerParams(
            dimension_semantics=("arbitrary",),
        ),
    )(x, W, b)


# speedup vs baseline: 1.0121x; 1.0121x over previous
import jax
import jax.numpy as jnp
from jax.experimental import pallas as pl
from jax.experimental.pallas import tpu as pltpu


D_MODEL = 4096
NUM_EXPERTS = 64
TILE_TOK = 512
X_BUFFERS = 4


def _outer(x_hbm, w_ref, b_ref, out_hbm):
    w = w_ref[...]
    bias = b_ref[...]

    def body(x_tile, out_tile):
        logits = jax.lax.dot_general(
            x_tile[...], w,
            dimension_numbers=(((1,), (1,)), ((), ())),
            preferred_element_type=jnp.float32,
        )
        logits = logits + bias
        m = jnp.max(logits, axis=-1, keepdims=True)
        e = jnp.exp(logits - m)
        out_tile[...] = e / jnp.sum(e, axis=-1, keepdims=True)

    n_tiles = x_hbm.shape[0] // TILE_TOK
    pipeline = pltpu.emit_pipeline(
        body,
        grid=(n_tiles,),
        in_specs=[
            pl.BlockSpec((TILE_TOK, D_MODEL), lambda i: (i, 0),
                         pipeline_mode=pl.Buffered(buffer_count=X_BUFFERS)),
        ],
        out_specs=[
            pl.BlockSpec((TILE_TOK, NUM_EXPERTS), lambda i: (i, 0)),
        ],
    )
    pipeline(x_hbm, out_hbm)


def kernel(x, W, b):
    n_tok = x.shape[0]
    return pl.pallas_call(
        _outer,
        in_specs=[
            pl.BlockSpec(memory_space=pltpu.MemorySpace.HBM),
            pl.BlockSpec(memory_space=pltpu.MemorySpace.VMEM),
            pl.BlockSpec(memory_space=pltpu.MemorySpace.VMEM),
        ],
        out_specs=pl.BlockSpec(memory_space=pltpu.MemorySpace.HBM),
        out_shape=jax.ShapeDtypeStruct((n_tok, NUM_EXPERTS), jnp.float32),
    )(x, W, b)
